# baseline (speedup 1.0000x reference)
import functools

import jax
import jax.numpy as jnp
from jax import lax
from jax.experimental import pallas as pl
from jax.experimental.pallas import tpu as pltpu
from jax.experimental.pallas import tpu_sc as plsc

N = 10000
NP = 10240
E = 320000
EP = 327680
C = 128
R = 80
NB = EP // (R * C)
D = 128
NC, NS = 2, 16
RN = N // NS
BLK = 1000


def _mesh():
    return plsc.VectorSubcoreMesh(core_axis_name="c", subcore_axis_name="s")


def _newton_rsqrt(x):
    i = lax.bitcast_convert_type(x, jnp.int32)
    i = jnp.full((16,), 0x5F3759DF, jnp.int32) - lax.shift_right_logical(i, 1)
    y = lax.bitcast_convert_type(i, jnp.float32)
    for _ in range(4):
        y = y * (1.5 - 0.5 * x * y * y)
    return y


@functools.partial(
    pl.kernel,
    out_type=(
        jax.ShapeDtypeStruct((NP,), jnp.float32),
        jax.ShapeDtypeStruct((NB, R, C), jnp.float32),
    ),
    mesh=_mesh(),
    compiler_params=pltpu.CompilerParams(needs_layout_passes=False),
    scratch_types=[
        pltpu.VMEM((2, R, C), jnp.int32),
        pltpu.VMEM((2, R, C), jnp.float32),
        pltpu.VMEM((R, C), jnp.int32),
        pltpu.VMEM((R, C), jnp.float32),
        pltpu.VMEM((NP,), jnp.float32),
        pltpu.VMEM_SHARED((NP,), jnp.float32),
    ],
)
def _prep(src_hbm, dst_hbm, w_hbm, ones_hbm, dinv_hbm, norm_hbm,
          dstv, wv, srcv, normv, degv, deg_sh):
    cid = lax.axis_index("c")
    sid = lax.axis_index("s")
    wid = cid * NS + sid

    @pl.when(sid == 0)
    def _():
        pltpu.sync_copy(ones_hbm, deg_sh)

    for p in range(2):
        pltpu.sync_copy(dst_hbm.at[2 * sid + p], dstv.at[p])
        pltpu.sync_copy(w_hbm.at[2 * sid + p], wv.at[p])
    plsc.subcore_barrier()

    def deg_body(j, carry):
        pltpu.sync_copy(wv.at[0, j], deg_sh.at[dstv.at[0, j]], add=True)
        pltpu.sync_copy(wv.at[1, j], deg_sh.at[dstv.at[1, j]], add=True)
        return carry
    lax.fori_loop(0, R, deg_body, 0)
    plsc.subcore_barrier()

    pltpu.sync_copy(deg_sh, degv)

    def rs_body(i, carry):
        degv[pl.ds(i * 16, 16)] = _newton_rsqrt(degv[pl.ds(i * 16, 16)])
        return carry
    lax.fori_loop(0, NP // 16, rs_body, 0)

    pltpu.sync_copy(src_hbm.at[wid], srcv)
    pltpu.sync_copy(dst_hbm.at[wid], dstv.at[0])
    pltpu.sync_copy(w_hbm.at[wid], wv.at[0])

    zero16 = jnp.zeros((16,), jnp.int32)

    def norm_body(t, carry):
        j = t // (C // 16)
        g = (t % (C // 16)) * 16
        jb = jnp.full((16,), j, jnp.int32)
        g16 = g + lax.iota(jnp.int32, 16)
        s16 = plsc.load_gather(srcv, [jb, g16])
        d16 = plsc.load_gather(dstv, [zero16, jb, g16])
        w16 = plsc.load_gather(wv, [zero16, jb, g16])
        ds_ = plsc.load_gather(degv, [s16])
        dd_ = plsc.load_gather(degv, [d16])
        plsc.store_scatter(normv, [jb, g16], ds_ * w16 * dd_)
        return carry
    lax.fori_loop(0, R * (C // 16), norm_body, 0)
    pltpu.sync_copy(normv, norm_hbm.at[wid])

    @pl.when(jnp.logical_and(cid == 0, sid == 0))
    def _():
        pltpu.sync_copy(degv, dinv_hbm)


ROWS_E = EP // C
R_SLOW, R_FAST = 0, 160


def _make_agg(edge_split):
    G = 8

    @functools.partial(
        pl.kernel,
        out_type=jax.ShapeDtypeStruct((2 * NS, RN, D), jnp.float32),
        mesh=_mesh(),
        compiler_params=pltpu.CompilerParams(needs_layout_passes=False),
        scratch_types=[
            pltpu.VMEM((2, G, C), jnp.int32),
            pltpu.VMEM((2, G, C), jnp.int32),
            pltpu.VMEM((2, G, C), jnp.float32),
            pltpu.VMEM((2, C, D), jnp.float32),
            pltpu.VMEM_SHARED((N, D), jnp.float32),
            pltpu.SemaphoreType.DMA,
            pltpu.SemaphoreType.DMA,
            pltpu.SemaphoreType.DMA,
            pltpu.SemaphoreType.DMA,
        ],
    )
    def _agg(tbl_hbm, gsrc_hbm, dst_hbm, norm_hbm, init_hbm, out_hbm,
             gidxv, dstv, normv, rows, acc, gsem0, gsem1, ssem0, ssem1):
        cid = lax.axis_index("c")
        sid = lax.axis_index("s")
        wid = cid * NS + sid
        gsem = (gsem0, gsem1)
        ssem = (ssem0, ssem1)

        pltpu.sync_copy(init_hbm.at[wid], acc.at[pl.ds(sid * RN, RN)])
        plsc.subcore_barrier()

        def pipeline(nch, ebase, gbase):
            def stage(k, slot):
                pltpu.sync_copy(gsrc_hbm.at[pl.ds(gbase + k * G, G)],
                                gidxv.at[slot])
                pltpu.sync_copy(dst_hbm.at[pl.ds(ebase + k * G, G)],
                                dstv.at[slot])
                pltpu.sync_copy(norm_hbm.at[pl.ds(ebase + k * G, G)],
                                normv.at[slot])

            stage(0, 0)
            pltpu.async_copy(tbl_hbm.at[gidxv.at[0, 0]], rows.at[0], gsem[0])

            def pair_body(h, carry):
                for bp in range(2):
                    g = h * 2 + bp
                    sg = (g // G) % 2
                    ig = g % G

                    @pl.when(g >= 1)
                    def _():
                        gp = g - 1
                        pltpu.make_async_copy(
                            rows.at[1 - bp],
                            acc.at[dstv.at[(gp // G) % 2, gp % G]],
                            ssem[1 - bp]).wait()

                    @pl.when(jnp.logical_and(ig == 0, g + G < nch))
                    def _():
                        stage(g // G + 1, 1 - sg)

                    @pl.when(g + 1 < nch)
                    def _():
                        sg1 = ((g + 1) // G) % 2
                        ig1 = (g + 1) % G
                        pltpu.async_copy(tbl_hbm.at[gidxv.at[sg1, ig1]],
                                         rows.at[1 - bp], gsem[1 - bp])

                    pltpu.make_async_copy(tbl_hbm.at[gidxv.at[sg, ig]],
                                          rows.at[bp], gsem[bp]).wait()
                    sg16 = jnp.full((16,), sg, jnp.int32)
                    ig16 = jnp.full((16,), ig, jnp.int32)

                    def row_body(r, rcarry):
                        nb = plsc.load_gather(
                            normv, [sg16, ig16, jnp.full((16,), r)])
                        for k in range(D // 16):
                            rows[bp, r, pl.ds(k * 16, 16)] = (
                                rows[bp, r, pl.ds(k * 16, 16)] * nb)
                        return rcarry
                    lax.fori_loop(0, C, row_body, 0)

                    pltpu.async_copy(rows.at[bp], acc.at[dstv.at[sg, ig]],
                                     ssem[bp], add=True)
                return carry
            lax.fori_loop(0, nch // 2, pair_body, 0)

            glast = nch - 1
            pltpu.make_async_copy(
                rows.at[glast % 2],
                acc.at[dstv.at[(glast // G) % 2, glast % G]],
                ssem[glast % 2]).wait()

        if edge_split:
            @pl.when(cid == 0)
            def _():
                base = pl.multiple_of(sid * R_FAST, G)
                pipeline(R_FAST, base, base)
        else:
            rpt = ROWS_E // NS
            ebase = pl.multiple_of(sid * rpt, G)
            gbase = pl.multiple_of(cid * ROWS_E + sid * rpt, G)
            pipeline(rpt, ebase, gbase)

        plsc.subcore_barrier()
        pltpu.sync_copy(acc.at[pl.ds(sid * RN, RN)], out_hbm.at[wid])

    return _agg


_agg_edge = _make_agg(True)
_agg_feat = _make_agg(False)


def _celu(v):
    return jnp.where(v > 0, v, jnp.exp(jnp.minimum(v, 0.0)) - 1.0)


def _dot(a, b):
    return jnp.dot(a, b, preferred_element_type=jnp.float32)


def _k0_body(x_ref, dv_ref, i0_ref):
    d2 = dv_ref[...] * dv_ref[...]
    i0_ref[0] = d2 * x_ref[...]
    i0_ref[1] = jnp.zeros_like(x_ref[...])


def _k1_body(p_ref, dv_ref, w1_ref, b1_ref, w2_ref, t2_ref, i2_ref):
    y = p_ref[0] + p_ref[1]
    h = _celu(_dot(y, w1_ref[...]) + b1_ref[...])
    t2 = _dot(h, w2_ref[...])
    d2 = dv_ref[...] * dv_ref[...]
    t2a = t2[:, :D]
    t2b = t2[:, D:]
    t2_ref[0] = t2a
    t2_ref[1] = t2b
    i2_ref[0] = d2 * t2a
    i2_ref[1] = d2 * t2b


def _k2_body(o2_ref, dv_ref, w3_ref, b2_ref, t3_ref, i3_ref):
    ha = _celu(o2_ref[0] + b2_ref[:, :D])
    hb = _celu(o2_ref[1] + b2_ref[:, D:])
    t3 = _dot(ha, w3_ref[:D]) + _dot(hb, w3_ref[D:])
    d2 = dv_ref[...] * dv_ref[...]
    t3_ref[...] = t3
    i3_ref[0] = d2 * t3
    i3_ref[1] = jnp.zeros_like(t3)


def _k3_body(p_ref, b3_ref, out_ref):
    out_ref[...] = _celu(p_ref[0] + p_ref[1] + b3_ref[...])


def _rows2(i):
    return (0, i, 0)


def _rows1(i):
    return (i, 0)


def _full2(i):
    return (0, 0)


_spec_stack = pl.BlockSpec((2, BLK, D), _rows2)
_spec_rows = pl.BlockSpec((BLK, D), _rows1)
_spec_dv = pl.BlockSpec((BLK, 1), _rows1)


def _k0(x, dv):
    return pl.pallas_call(
        _k0_body,
        grid=(N // BLK,),
        in_specs=[_spec_rows, _spec_dv],
        out_specs=_spec_stack,
        out_shape=jax.ShapeDtypeStruct((2, N, D), jnp.float32),
    )(x, dv)


def _k1(p, dv, W1, b1, W2):
    return pl.pallas_call(
        _k1_body,
        grid=(N // BLK,),
        in_specs=[
            _spec_stack,
            _spec_dv,
            pl.BlockSpec((128, 256), _full2),
            pl.BlockSpec((1, 256), _full2),
            pl.BlockSpec((256, 256), _full2),
        ],
        out_specs=[_spec_stack, _spec_stack],
        out_shape=[
            jax.ShapeDtypeStruct((2, N, D), jnp.float32),
            jax.ShapeDtypeStruct((2, N, D), jnp.float32),
        ],
    )(p, dv, W1, b1, W2)


def _k2(o2, dv, W3, b2):
    return pl.pallas_call(
        _k2_body,
        grid=(N // BLK,),
        in_specs=[
            _spec_stack,
            _spec_dv,
            pl.BlockSpec((256, 128), _full2),
            pl.BlockSpec((1, 256), _full2),
        ],
        out_specs=[_spec_rows, _spec_stack],
        out_shape=[
            jax.ShapeDtypeStruct((N, D), jnp.float32),
            jax.ShapeDtypeStruct((2, N, D), jnp.float32),
        ],
    )(o2, dv, W3, b2)


def _k3(p, b3):
    return pl.pallas_call(
        _k3_body,
        grid=(N // BLK,),
        in_specs=[_spec_stack, pl.BlockSpec((1, 128), _full2)],
        out_specs=_spec_rows,
        out_shape=jax.ShapeDtypeStruct((N, D), jnp.float32),
    )(p, b3)


def kernel(x, edge_index, edge_weight, W1, b1, W2, b2, W3, b3):
    pad = jnp.zeros((EP - E,), jnp.int32)
    src = jnp.concatenate([edge_index[0].astype(jnp.int32), pad])
    dst = jnp.concatenate([edge_index[1].astype(jnp.int32), pad])
    w = jnp.concatenate([edge_weight.astype(jnp.float32),
                         jnp.zeros((EP - E,), jnp.float32)])
    src3 = src.reshape(NB, R, C)
    dst3 = dst.reshape(NB, R, C)
    w3 = w.reshape(NB, R, C)
    src2 = src.reshape(ROWS_E, C)
    dst2 = dst.reshape(ROWS_E, C)
    gsrc2 = jnp.concatenate([src, src + N]).reshape(2 * ROWS_E, C)
    ones = jnp.ones((NP,), jnp.float32)

    dinv, norm3 = _prep(src3, dst3, w3, ones)
    dv = dinv[:N].reshape(N, 1)

    i0 = _k0(x, dv)
    norm2 = norm3.reshape(ROWS_E, C)
    p1 = _agg_edge(x, src2, dst2, norm2, i0.reshape(2 * NS, RN, D))
    t2, i2 = _k1(p1.reshape(2, N, D), dv, W1, b1.reshape(1, -1), W2)
    o2 = _agg_feat(t2.reshape(2 * N, D), gsrc2, dst2, norm2,
                   i2.reshape(2 * NS, RN, D))
    t3, i3 = _k2(o2.reshape(2, N, D), dv, W3, b2.reshape(1, -1))
    p3 = _agg_edge(t3, src2, dst2, norm2, i3.reshape(2 * NS, RN, D))
    return _k3(p3.reshape(2, N, D), b3.reshape(1, -1))


# speedup vs baseline: 1.3746x; 1.3746x over previous
import functools

import jax
import jax.numpy as jnp
from jax import lax
from jax.experimental import pallas as pl
from jax.experimental.pallas import tpu as pltpu
from jax.experimental.pallas import tpu_sc as plsc

N = 10000
NP = 10240
E = 320000
EP = 327680
C = 128
R = 80
NB = EP // (R * C)
D = 128
NC, NS = 2, 16
RN = N // NS
BLK = 1000


def _mesh():
    return plsc.VectorSubcoreMesh(core_axis_name="c", subcore_axis_name="s")


def _newton_rsqrt(x):
    i = lax.bitcast_convert_type(x, jnp.int32)
    i = jnp.full((16,), 0x5F3759DF, jnp.int32) - lax.shift_right_logical(i, 1)
    y = lax.bitcast_convert_type(i, jnp.float32)
    for _ in range(4):
        y = y * (1.5 - 0.5 * x * y * y)
    return y


@functools.partial(
    pl.kernel,
    out_type=(
        jax.ShapeDtypeStruct((NP,), jnp.float32),
        jax.ShapeDtypeStruct((NB, R, C), jnp.float32),
    ),
    mesh=_mesh(),
    compiler_params=pltpu.CompilerParams(needs_layout_passes=False),
    scratch_types=[
        pltpu.VMEM((2, R, C), jnp.int32),
        pltpu.VMEM((2, R, C), jnp.float32),
        pltpu.VMEM((R, C), jnp.int32),
        pltpu.VMEM((R, C), jnp.float32),
        pltpu.VMEM((NP,), jnp.float32),
        pltpu.VMEM_SHARED((NP,), jnp.float32),
    ],
)
def _prep(src_hbm, dst_hbm, w_hbm, ones_hbm, dinv_hbm, norm_hbm,
          dstv, wv, srcv, normv, degv, deg_sh):
    cid = lax.axis_index("c")
    sid = lax.axis_index("s")
    wid = cid * NS + sid

    @pl.when(sid == 0)
    def _():
        pltpu.sync_copy(ones_hbm, deg_sh)

    for p in range(2):
        pltpu.sync_copy(dst_hbm.at[2 * sid + p], dstv.at[p])
        pltpu.sync_copy(w_hbm.at[2 * sid + p], wv.at[p])
    plsc.subcore_barrier()

    def deg_body(j, carry):
        pltpu.sync_copy(wv.at[0, j], deg_sh.at[dstv.at[0, j]], add=True)
        pltpu.sync_copy(wv.at[1, j], deg_sh.at[dstv.at[1, j]], add=True)
        return carry
    lax.fori_loop(0, R, deg_body, 0)
    plsc.subcore_barrier()

    pltpu.sync_copy(deg_sh, degv)

    def rs_body(i, carry):
        degv[pl.ds(i * 16, 16)] = _newton_rsqrt(degv[pl.ds(i * 16, 16)])
        return carry
    lax.fori_loop(0, NP // 16, rs_body, 0)

    pltpu.sync_copy(src_hbm.at[wid], srcv)
    pltpu.sync_copy(dst_hbm.at[wid], dstv.at[0])
    pltpu.sync_copy(w_hbm.at[wid], wv.at[0])

    zero16 = jnp.zeros((16,), jnp.int32)

    def norm_body(t, carry):
        j = t // (C // 16)
        g = (t % (C // 16)) * 16
        jb = jnp.full((16,), j, jnp.int32)
        g16 = g + lax.iota(jnp.int32, 16)
        s16 = plsc.load_gather(srcv, [jb, g16])
        d16 = plsc.load_gather(dstv, [zero16, jb, g16])
        w16 = plsc.load_gather(wv, [zero16, jb, g16])
        ds_ = plsc.load_gather(degv, [s16])
        dd_ = plsc.load_gather(degv, [d16])
        plsc.store_scatter(normv, [jb, g16], ds_ * w16 * dd_)
        return carry
    lax.fori_loop(0, R * (C // 16), norm_body, 0)
    pltpu.sync_copy(normv, norm_hbm.at[wid])

    @pl.when(jnp.logical_and(cid == 0, sid == 0))
    def _():
        pltpu.sync_copy(degv, dinv_hbm)


ROWS_E = EP // C
R_SLOW, R_FAST = 8, 152


def _make_agg(edge_split):
    G = 8

    @functools.partial(
        pl.kernel,
        out_type=jax.ShapeDtypeStruct((2 * NS, RN, D), jnp.float32),
        mesh=_mesh(),
        compiler_params=pltpu.CompilerParams(needs_layout_passes=False),
        scratch_types=[
            pltpu.VMEM((2, G, C), jnp.int32),
            pltpu.VMEM((2, G, C), jnp.int32),
            pltpu.VMEM((2, G, C), jnp.float32),
            pltpu.VMEM((2, C, D), jnp.float32),
            pltpu.VMEM_SHARED((N, D), jnp.float32),
            pltpu.SemaphoreType.DMA,
            pltpu.SemaphoreType.DMA,
            pltpu.SemaphoreType.DMA,
            pltpu.SemaphoreType.DMA,
        ],
    )
    def _agg(tbl_hbm, gsrc_hbm, dst_hbm, norm_hbm, init_hbm, out_hbm,
             gidxv, dstv, normv, rows, acc, gsem0, gsem1, ssem0, ssem1):
        cid = lax.axis_index("c")
        sid = lax.axis_index("s")
        wid = cid * NS + sid
        gsem = (gsem0, gsem1)
        ssem = (ssem0, ssem1)

        pltpu.sync_copy(init_hbm.at[wid], acc.at[pl.ds(sid * RN, RN)])
        plsc.subcore_barrier()

        def pipeline(nch, ebase, gbase):
            def stage(k, slot):
                pltpu.sync_copy(gsrc_hbm.at[pl.ds(gbase + k * G, G)],
                                gidxv.at[slot])
                pltpu.sync_copy(dst_hbm.at[pl.ds(ebase + k * G, G)],
                                dstv.at[slot])
                pltpu.sync_copy(norm_hbm.at[pl.ds(ebase + k * G, G)],
                                normv.at[slot])

            stage(0, 0)
            pltpu.async_copy(tbl_hbm.at[gidxv.at[0, 0]], rows.at[0], gsem[0])

            def pair_body(h, carry):
                for bp in range(2):
                    g = h * 2 + bp
                    sg = (g // G) % 2
                    ig = g % G

                    @pl.when(g >= 1)
                    def _():
                        gp = g - 1
                        pltpu.make_async_copy(
                            rows.at[1 - bp],
                            acc.at[dstv.at[(gp // G) % 2, gp % G]],
                            ssem[1 - bp]).wait()

                    @pl.when(jnp.logical_and(ig == 0, g + G < nch))
                    def _():
                        stage(g // G + 1, 1 - sg)

                    @pl.when(g + 1 < nch)
                    def _():
                        sg1 = ((g + 1) // G) % 2
                        ig1 = (g + 1) % G
                        pltpu.async_copy(tbl_hbm.at[gidxv.at[sg1, ig1]],
                                         rows.at[1 - bp], gsem[1 - bp])

                    pltpu.make_async_copy(tbl_hbm.at[gidxv.at[sg, ig]],
                                          rows.at[bp], gsem[bp]).wait()
                    sg16 = jnp.full((16,), sg, jnp.int32)
                    ig16 = jnp.full((16,), ig, jnp.int32)

                    def row_body(r, rcarry):
                        nb = plsc.load_gather(
                            normv, [sg16, ig16, jnp.full((16,), r)])
                        for k in range(D // 16):
                            rows[bp, r, pl.ds(k * 16, 16)] = (
                                rows[bp, r, pl.ds(k * 16, 16)] * nb)
                        return rcarry
                    lax.fori_loop(0, C, row_body, 0)

                    pltpu.async_copy(rows.at[bp], acc.at[dstv.at[sg, ig]],
                                     ssem[bp], add=True)
                return carry
            lax.fori_loop(0, nch // 2, pair_body, 0)

            glast = nch - 1
            pltpu.make_async_copy(
                rows.at[glast % 2],
                acc.at[dstv.at[(glast // G) % 2, glast % G]],
                ssem[glast % 2]).wait()

        if edge_split:
            @pl.when(cid == 0)
            def _():
                base = pl.multiple_of(sid * R_FAST, G)
                pipeline(R_FAST, base, base)

            @pl.when(cid == 1)
            def _():
                base = pl.multiple_of(NS * R_FAST + sid * R_SLOW, G)
                pipeline(R_SLOW, base, base)
        else:
            rpt = ROWS_E // NS
            ebase = pl.multiple_of(sid * rpt, G)
            gbase = pl.multiple_of(cid * ROWS_E + sid * rpt, G)
            pipeline(rpt, ebase, gbase)

        plsc.subcore_barrier()
        pltpu.sync_copy(acc.at[pl.ds(sid * RN, RN)], out_hbm.at[wid])

    return _agg


_agg_edge = _make_agg(True)
_agg_feat = _make_agg(False)


def _celu(v):
    return jnp.where(v > 0, v, jnp.exp(jnp.minimum(v, 0.0)) - 1.0)


def _dot(a, b):
    return jnp.dot(a, b, preferred_element_type=jnp.float32)


def _k0_body(x_ref, dv_ref, i0_ref):
    d2 = dv_ref[...] * dv_ref[...]
    i0_ref[0] = d2 * x_ref[...]
    i0_ref[1] = jnp.zeros_like(x_ref[...])


def _k1_body(p_ref, dv_ref, w1_ref, b1_ref, w2_ref, t2_ref, i2_ref):
    y = p_ref[0] + p_ref[1]
    h = _celu(_dot(y, w1_ref[...]) + b1_ref[...])
    t2 = _dot(h, w2_ref[...])
    d2 = dv_ref[...] * dv_ref[...]
    t2a = t2[:, :D]
    t2b = t2[:, D:]
    t2_ref[0] = t2a
    t2_ref[1] = t2b
    i2_ref[0] = d2 * t2a
    i2_ref[1] = d2 * t2b


def _k2_body(o2_ref, dv_ref, w3_ref, b2_ref, t3_ref, i3_ref):
    ha = _celu(o2_ref[0] + b2_ref[:, :D])
    hb = _celu(o2_ref[1] + b2_ref[:, D:])
    t3 = _dot(ha, w3_ref[:D]) + _dot(hb, w3_ref[D:])
    d2 = dv_ref[...] * dv_ref[...]
    t3_ref[...] = t3
    i3_ref[0] = d2 * t3
    i3_ref[1] = jnp.zeros_like(t3)


def _k3_body(p_ref, b3_ref, out_ref):
    out_ref[...] = _celu(p_ref[0] + p_ref[1] + b3_ref[...])


def _rows2(i):
    return (0, i, 0)


def _rows1(i):
    return (i, 0)


def _full2(i):
    return (0, 0)


_spec_stack = pl.BlockSpec((2, BLK, D), _rows2)
_spec_rows = pl.BlockSpec((BLK, D), _rows1)
_spec_dv = pl.BlockSpec((BLK, 1), _rows1)


def _k0(x, dv):
    return pl.pallas_call(
        _k0_body,
        grid=(N // BLK,),
        in_specs=[_spec_rows, _spec_dv],
        out_specs=_spec_stack,
        out_shape=jax.ShapeDtypeStruct((2, N, D), jnp.float32),
    )(x, dv)


def _k1(p, dv, W1, b1, W2):
    return pl.pallas_call(
        _k1_body,
        grid=(N // BLK,),
        in_specs=[
            _spec_stack,
            _spec_dv,
            pl.BlockSpec((128, 256), _full2),
            pl.BlockSpec((1, 256), _full2),
            pl.BlockSpec((256, 256), _full2),
        ],
        out_specs=[_spec_stack, _spec_stack],
        out_shape=[
            jax.ShapeDtypeStruct((2, N, D), jnp.float32),
            jax.ShapeDtypeStruct((2, N, D), jnp.float32),
        ],
    )(p, dv, W1, b1, W2)


def _k2(o2, dv, W3, b2):
    return pl.pallas_call(
        _k2_body,
        grid=(N // BLK,),
        in_specs=[
            _spec_stack,
            _spec_dv,
            pl.BlockSpec((256, 128), _full2),
            pl.BlockSpec((1, 256), _full2),
        ],
        out_specs=[_spec_rows, _spec_stack],
        out_shape=[
            jax.ShapeDtypeStruct((N, D), jnp.float32),
            jax.ShapeDtypeStruct((2, N, D), jnp.float32),
        ],
    )(o2, dv, W3, b2)


def _k3(p, b3):
    return pl.pallas_call(
        _k3_body,
        grid=(N // BLK,),
        in_specs=[_spec_stack, pl.BlockSpec((1, 128), _full2)],
        out_specs=_spec_rows,
        out_shape=jax.ShapeDtypeStruct((N, D), jnp.float32),
    )(p, b3)


def kernel(x, edge_index, edge_weight, W1, b1, W2, b2, W3, b3):
    pad = jnp.zeros((EP - E,), jnp.int32)
    src = jnp.concatenate([edge_index[0].astype(jnp.int32), pad])
    dst = jnp.concatenate([edge_index[1].astype(jnp.int32), pad])
    w = jnp.concatenate([edge_weight.astype(jnp.float32),
                         jnp.zeros((EP - E,), jnp.float32)])
    src3 = src.reshape(NB, R, C)
    dst3 = dst.reshape(NB, R, C)
    w3 = w.reshape(NB, R, C)
    src2 = src.reshape(ROWS_E, C)
    dst2 = dst.reshape(ROWS_E, C)
    gsrc2 = jnp.concatenate([src, src + N]).reshape(2 * ROWS_E, C)
    ones = jnp.ones((NP,), jnp.float32)

    dinv, norm3 = _prep(src3, dst3, w3, ones)
    dv = dinv[:N].reshape(N, 1)

    i0 = _k0(x, dv)
    norm2 = norm3.reshape(ROWS_E, C)
    p1 = _agg_edge(x, src2, dst2, norm2, i0.reshape(2 * NS, RN, D))
    t2, i2 = _k1(p1.reshape(2, N, D), dv, W1, b1.reshape(1, -1), W2)
    o2 = _agg_feat(t2.reshape(2 * N, D), gsrc2, dst2, norm2,
                   i2.reshape(2 * NS, RN, D))
    t3, i3 = _k2(o2.reshape(2, N, D), dv, W3, b2.reshape(1, -1))
    p3 = _agg_edge(t3, src2, dst2, norm2, i3.reshape(2 * NS, RN, D))
    return _k3(p3.reshape(2, N, D), b3.reshape(1, -1))
